# baseline (speedup 1.0000x reference)
import functools

import jax
import jax.numpy as jnp
from jax import lax
from jax.experimental import pallas as pl
from jax.experimental.pallas import tpu as pltpu
from jax.experimental.pallas import tpu_sc as plsc

N = 10000
E = 320000
H = 128

NC = 2
NS = 16
NW = NC * NS
EPW = E // NW
C = 80
NCHUNK = EPW // C
RPS = 640

BE = 1000
BN = 1000

_EPS = 1e-5


def _gelu(x):
    z = x * 0.7071067811865476
    az = jnp.abs(z)
    t = 1.0 / (1.0 + 0.3275911 * az)
    poly = t * (0.254829592
                + t * (-0.284496736
                       + t * (1.421413741
                              + t * (-1.453152027 + t * 1.061405429))))
    erf_abs = 1.0 - poly * jnp.exp(-az * az)
    erf = jnp.where(z >= 0.0, erf_abs, -erf_abs)
    return 0.5 * x * (1.0 + erf)


def _gelu_sc(x):
    x2 = x * x
    e = jnp.exp(x * (1.5957691216057308 + 0.07135481627442266 * x2))
    return x - x / (1.0 + e)


def _mm(a, w):
    return lax.dot_general(a, w, (((1,), (1,)), ((), ())),
                           preferred_element_type=jnp.float32)


def _mmb(a, w):
    return lax.dot_general(a.astype(jnp.bfloat16), w.astype(jnp.bfloat16),
                           (((1,), (1,)), ((), ())),
                           preferred_element_type=jnp.float32)


def _ln(x, g, b):
    m = jnp.mean(x, axis=1, keepdims=True)
    v = jnp.mean(jnp.square(x - m), axis=1, keepdims=True)
    return (x - m) * lax.rsqrt(v + _EPS) * g + b


def _full(shape):
    return pl.BlockSpec(shape, lambda i: tuple(0 for _ in shape))


def _edge_prep_body(et, g, b, w, bias, w0, w1, o0, o1):
    h = _gelu(_mm(_ln(et[...], g[...], b[...]), w[...]) + bias[...])
    o0[...] = _mm(h, w0[...])
    o1[...] = _mm(h, w1[...])


def _edge_prep(edge_tokens, ep_g, ep_b, ep_W, ep_bias, W0, W1):
    return pl.pallas_call(
        _edge_prep_body,
        grid=(E // BE,),
        in_specs=[pl.BlockSpec((BE, H), lambda i: (i, 0)),
                  _full((1, H)), _full((1, H)), _full((H, H)), _full((1, H)),
                  _full((H, H)), _full((H, H))],
        out_specs=[pl.BlockSpec((BE, H), lambda i: (i, 0))] * 2,
        out_shape=[jax.ShapeDtypeStruct((E, H), jnp.float32)] * 2,
    )(edge_tokens, ep_g.reshape(1, H), ep_b.reshape(1, H), ep_W,
      ep_bias.reshape(1, H), W0, W1)


def _node_prep_body(nt, g, b, w, bias, wsrc, oh, oxs):
    h = _gelu(_mm(_ln(nt[...], g[...], b[...]), w[...]) + bias[...])
    oh[...] = h
    oxs[...] = _mm(h, wsrc[...])


def _node_prep(node_tokens, np_g, np_b, np_W, np_bias, Wsrc0):
    return pl.pallas_call(
        _node_prep_body,
        grid=(N // BN,),
        in_specs=[pl.BlockSpec((BN, H), lambda i: (i, 0)),
                  _full((1, H)), _full((1, H)), _full((H, H)), _full((1, H)),
                  _full((H, H))],
        out_specs=[pl.BlockSpec((BN, H), lambda i: (i, 0))] * 2,
        out_shape=[jax.ShapeDtypeStruct((N, H), jnp.float32)] * 2,
    )(node_tokens, np_g.reshape(1, H), np_b.reshape(1, H), np_W,
      np_bias.reshape(1, H), Wsrc0)


def _node_update_body_next(x, a0, a1, d0, d1, wo, g, b, wn, out, oxs):
    deg = jnp.sum(d0[...] + d1[...], axis=1, keepdims=True) * (1.0 / H)
    deg = jnp.maximum(deg, 1.0)
    y = x[...] + _mm((a0[...] + a1[...]) / deg, wo[...])
    o = _ln(y, g[...], b[...])
    out[...] = o
    oxs[...] = _mm(o, wn[...])


def _node_update_body_last(x, a0, a1, d0, d1, wo, g, b, out):
    deg = jnp.sum(d0[...] + d1[...], axis=1, keepdims=True) * (1.0 / H)
    deg = jnp.maximum(deg, 1.0)
    y = x[...] + _mm((a0[...] + a1[...]) / deg, wo[...])
    out[...] = _ln(y, g[...], b[...])


def _node_update(x, a0, a1, d0, d1, Wout, g, b, Wnext):
    ins = [pl.BlockSpec((BN, H), lambda i: (i, 0))] * 5 + \
          [_full((H, H)), _full((1, H)), _full((1, H))]
    args = [x, a0, a1, d0, d1, Wout, g.reshape(1, H), b.reshape(1, H)]
    if Wnext is not None:
        return pl.pallas_call(
            _node_update_body_next,
            grid=(N // BN,),
            in_specs=ins + [_full((H, H))],
            out_specs=[pl.BlockSpec((BN, H), lambda i: (i, 0))] * 2,
            out_shape=[jax.ShapeDtypeStruct((N, H), jnp.float32)] * 2,
        )(*args, Wnext)
    return pl.pallas_call(
        _node_update_body_last,
        grid=(N // BN,),
        in_specs=ins,
        out_specs=pl.BlockSpec((BN, H), lambda i: (i, 0)),
        out_shape=jax.ShapeDtypeStruct((N, H), jnp.float32),
    )(*args)


def _zero_vec_buf(buf, rows):
    zero = jnp.zeros((16,), jnp.float32)
    cols = buf.shape[1] // 16

    def zrow(r, _):
        for k in range(cols):
            buf[r, pl.ds(k * 16, 16)] = zero
        return 0

    lax.fori_loop(0, rows, zrow, 0)


def _spmem_partition(s):
    base = s * RPS
    nown = jnp.where(s == NS - 1, (N - (NS - 1) * RPS) // C, RPS // C)
    return base, nown


@functools.lru_cache(maxsize=None)
def _sc_layer_kernel():
    mesh = plsc.VectorSubcoreMesh(core_axis_name="c", subcore_axis_name="s")
    NB = 3
    scratch = (
        [pltpu.VMEM((2, C), jnp.int32) for _ in range(NB)]
        + [pltpu.VMEM((C, H), jnp.float32) for _ in range(NB)]
        + [pltpu.VMEM_SHARED((N, H), jnp.float32)]
        + [pltpu.SemaphoreType.DMA] * (2 * NB)
    )

    def body(xs_hbm, eh_hbm, idx_hbm, agg_out, *rest):
        idxs = rest[0:NB]
        ehbs = rest[NB:2 * NB]
        agg_sh = rest[2 * NB]
        sems = rest[2 * NB + 1:]
        slots = [(idxs[b], ehbs[b], sems[2 * b], sems[2 * b + 1])
                 for b in range(NB)]
        c = lax.axis_index("c")
        s = lax.axis_index("s")
        wid = c * NS + s
        base, nown = _spmem_partition(s)

        _zero_vec_buf(ehbs[0], C)

        def zchunk(j, _):
            pltpu.sync_copy(ehbs[0], agg_sh.at[pl.ds(base + j * C, C)])
            return 0

        lax.fori_loop(0, nown, zchunk, 0)
        plsc.subcore_barrier()

        cbase = wid * NCHUNK

        def start(t, sl):
            idxv, ehb, se, sx = sl
            pltpu.sync_copy(idx_hbm.at[cbase + t], idxv)
            pltpu.async_copy(eh_hbm.at[pl.ds((cbase + t) * C, C)], ehb, se)

        def gather(t, sl):
            idxv, ehb, se, sx = sl
            pltpu.make_async_copy(eh_hbm.at[pl.ds(0, C)], ehb, se).wait()
            pltpu.async_copy(xs_hbm.at[idxv.at[0]], ehb, sx, add=True)

        def finish(t, sl, nxt_sl, nxt2_sl):
            idxv, ehb, se, sx = sl
            pltpu.make_async_copy(xs_hbm.at[idxv.at[0]], ehb, sx).wait()

            @pl.when(t + 2 < NCHUNK)
            def _():
                start(t + 2, nxt2_sl)

            @pl.when(t + 1 < NCHUNK)
            def _():
                gather(t + 1, nxt_sl)

            @plsc.parallel_loop(0, C, unroll=8)
            def _row(r):
                for k in range(H // 16):
                    cs = pl.ds(k * 16, 16)
                    ehb[r, cs] = _gelu_sc(ehb[r, cs])

            pltpu.sync_copy(ehb, agg_sh.at[idxv.at[1]], add=True)

        start(0, slots[0])
        start(1, slots[1])
        gather(0, slots[0])

        @pl.loop(0, NCHUNK, step=NB)
        def _main(i3):
            for b in range(NB):
                @pl.when(i3 + b < NCHUNK)
                def _(b=b):
                    finish(i3 + b, slots[b], slots[(b + 1) % NB],
                           slots[(b + 2) % NB])

        plsc.subcore_barrier()

        obase = c * N + base

        def wchunk(j, _):
            pltpu.sync_copy(agg_sh.at[pl.ds(base + j * C, C)], ehbs[0])
            pltpu.sync_copy(ehbs[0], agg_out.at[pl.ds(obase + j * C, C)])
            return 0

        lax.fori_loop(0, nown, wchunk, 0)

    return pl.kernel(
        body, mesh=mesh,
        out_type=[jax.ShapeDtypeStruct((NC * N, H), jnp.float32)],
        scratch_types=scratch)


@functools.lru_cache(maxsize=None)
def _sc_deg_kernel():
    mesh = plsc.VectorSubcoreMesh(core_axis_name="c", subcore_axis_name="s")
    scratch = [
        pltpu.VMEM((C,), jnp.int32),
        pltpu.VMEM((C, H), jnp.float32),
        pltpu.VMEM_SHARED((N, H), jnp.float32),
    ]

    def body(dst_hbm, deg_out, dstv, onesb, deg_sh):
        c = lax.axis_index("c")
        s = lax.axis_index("s")
        wid = c * NS + s
        base, nown = _spmem_partition(s)

        _zero_vec_buf(onesb, C)

        def zchunk(j, _):
            pltpu.sync_copy(onesb, deg_sh.at[pl.ds(base + j * C, C)])
            return 0

        lax.fori_loop(0, nown, zchunk, 0)
        one = jnp.ones((16,), jnp.float32)

        def orow(r, _):
            for k in range(H // 16):
                onesb[r, pl.ds(k * 16, 16)] = one
            return 0

        lax.fori_loop(0, C, orow, 0)
        plsc.subcore_barrier()

        ebase = wid * EPW

        def chunk(i, _):
            off = ebase + i * C
            pltpu.sync_copy(dst_hbm.at[pl.ds(off, C)], dstv)
            pltpu.sync_copy(onesb, deg_sh.at[dstv], add=True)
            return 0

        lax.fori_loop(0, NCHUNK, chunk, 0)
        plsc.subcore_barrier()

        obase = c * N + base

        def wchunk(j, _):
            pltpu.sync_copy(deg_sh.at[pl.ds(base + j * C, C)], onesb)
            pltpu.sync_copy(onesb, deg_out.at[pl.ds(obase + j * C, C)])
            return 0

        lax.fori_loop(0, nown, wchunk, 0)

    return pl.kernel(
        body, mesh=mesh,
        out_type=[jax.ShapeDtypeStruct((NC * N, H), jnp.float32)],
        scratch_types=scratch)


def kernel(node_tokens, edge_tokens, question_tokens, edge_index, edge_batch,
           node_ptr, start_node_locals, start_ptr,
           np_g, np_b, np_W, np_bias, ep_g, ep_b, ep_W, ep_bias,
           l0_Wsrc, l0_Wedge, l0_Wout, l0_g, l0_b,
           l1_Wsrc, l1_Wedge, l1_Wout, l1_g, l1_b):
    src = edge_index[0]
    dst = edge_index[1]
    idxp = jnp.stack([src.reshape(NW, NCHUNK, C), dst.reshape(NW, NCHUNK, C)],
                     axis=2).reshape(NW * NCHUNK, 2, C)

    eh0, eh1 = _edge_prep(edge_tokens, ep_g, ep_b, ep_W, ep_bias,
                          l0_Wedge, l1_Wedge)
    node_h, xs0 = _node_prep(node_tokens, np_g, np_b, np_W, np_bias, l0_Wsrc)

    (degf,) = _sc_deg_kernel()(dst)
    (aggf,) = _sc_layer_kernel()(xs0, eh0, idxp)
    x1, xs1 = _node_update(node_h, aggf[:N], aggf[N:], degf[:N], degf[N:],
                           l0_Wout, l0_g, l0_b, l1_Wsrc)

    (aggf1,) = _sc_layer_kernel()(xs1, eh1, idxp)
    x2 = _node_update(x1, aggf1[:N], aggf1[N:], degf[:N], degf[N:],
                      l1_Wout, l1_g, l1_b, None)
    return x2


# speedup vs baseline: 1.4090x; 1.4090x over previous
import functools

import jax
import jax.numpy as jnp
from jax import lax
from jax.experimental import pallas as pl
from jax.experimental.pallas import tpu as pltpu
from jax.experimental.pallas import tpu_sc as plsc

N = 10000
E = 320000
H = 128

NC = 2
NS = 16
NW = NC * NS
EPW = E // NW
C = 80
NCHUNK = EPW // C
RPS = 640

BE = 1000
BN = 1000

_EPS = 1e-5


def _gelu(x):
    z = x * 0.7071067811865476
    az = jnp.abs(z)
    t = 1.0 / (1.0 + 0.3275911 * az)
    poly = t * (0.254829592
                + t * (-0.284496736
                       + t * (1.421413741
                              + t * (-1.453152027 + t * 1.061405429))))
    erf_abs = 1.0 - poly * jnp.exp(-az * az)
    erf = jnp.where(z >= 0.0, erf_abs, -erf_abs)
    return 0.5 * x * (1.0 + erf)


def _gelu_sc(x):
    x2 = x * x
    e = jnp.exp(x * (1.5957691216057308 + 0.07135481627442266 * x2))
    return x - x / (1.0 + e)


def _mm(a, w):
    return lax.dot_general(a, w, (((1,), (1,)), ((), ())),
                           preferred_element_type=jnp.float32)


def _mmb(a, w):
    return lax.dot_general(a.astype(jnp.bfloat16), w.astype(jnp.bfloat16),
                           (((1,), (1,)), ((), ())),
                           preferred_element_type=jnp.float32)


def _ln(x, g, b):
    m = jnp.mean(x, axis=1, keepdims=True)
    v = jnp.mean(jnp.square(x - m), axis=1, keepdims=True)
    return (x - m) * lax.rsqrt(v + _EPS) * g + b


def _full(shape):
    return pl.BlockSpec(shape, lambda i: tuple(0 for _ in shape))


def _edge_prep_body(et, g, b, w, bias, w0, w1, o0, o1):
    h = _gelu(_mm(_ln(et[...], g[...], b[...]), w[...]) + bias[...])
    o0[...] = _mm(h, w0[...])
    o1[...] = _mm(h, w1[...])


def _edge_prep(edge_tokens, ep_g, ep_b, ep_W, ep_bias, W0, W1):
    return pl.pallas_call(
        _edge_prep_body,
        grid=(E // BE,),
        in_specs=[pl.BlockSpec((BE, H), lambda i: (i, 0)),
                  _full((1, H)), _full((1, H)), _full((H, H)), _full((1, H)),
                  _full((H, H)), _full((H, H))],
        out_specs=[pl.BlockSpec((BE, H), lambda i: (i, 0))] * 2,
        out_shape=[jax.ShapeDtypeStruct((E, H), jnp.float32)] * 2,
    )(edge_tokens, ep_g.reshape(1, H), ep_b.reshape(1, H), ep_W,
      ep_bias.reshape(1, H), W0, W1)


def _node_prep_body(nt, g, b, w, bias, wsrc, oh, oxs):
    h = _gelu(_mm(_ln(nt[...], g[...], b[...]), w[...]) + bias[...])
    oh[...] = h
    oxs[...] = _mm(h, wsrc[...])


def _node_prep(node_tokens, np_g, np_b, np_W, np_bias, Wsrc0):
    return pl.pallas_call(
        _node_prep_body,
        grid=(N // BN,),
        in_specs=[pl.BlockSpec((BN, H), lambda i: (i, 0)),
                  _full((1, H)), _full((1, H)), _full((H, H)), _full((1, H)),
                  _full((H, H))],
        out_specs=[pl.BlockSpec((BN, H), lambda i: (i, 0))] * 2,
        out_shape=[jax.ShapeDtypeStruct((N, H), jnp.float32)] * 2,
    )(node_tokens, np_g.reshape(1, H), np_b.reshape(1, H), np_W,
      np_bias.reshape(1, H), Wsrc0)


def _node_update_body_next(x, a0, a1, d0, d1, wo, g, b, wn, out, oxs):
    deg = jnp.sum(d0[...] + d1[...], axis=1, keepdims=True) * (1.0 / H)
    deg = jnp.maximum(deg, 1.0)
    y = x[...] + _mm((a0[...] + a1[...]) / deg, wo[...])
    o = _ln(y, g[...], b[...])
    out[...] = o
    oxs[...] = _mm(o, wn[...])


def _node_update_body_last(x, a0, a1, d0, d1, wo, g, b, out):
    deg = jnp.sum(d0[...] + d1[...], axis=1, keepdims=True) * (1.0 / H)
    deg = jnp.maximum(deg, 1.0)
    y = x[...] + _mm((a0[...] + a1[...]) / deg, wo[...])
    out[...] = _ln(y, g[...], b[...])


def _node_update(x, a0, a1, d0, d1, Wout, g, b, Wnext):
    ins = [pl.BlockSpec((BN, H), lambda i: (i, 0))] * 5 + \
          [_full((H, H)), _full((1, H)), _full((1, H))]
    args = [x, a0, a1, d0, d1, Wout, g.reshape(1, H), b.reshape(1, H)]
    if Wnext is not None:
        return pl.pallas_call(
            _node_update_body_next,
            grid=(N // BN,),
            in_specs=ins + [_full((H, H))],
            out_specs=[pl.BlockSpec((BN, H), lambda i: (i, 0))] * 2,
            out_shape=[jax.ShapeDtypeStruct((N, H), jnp.float32)] * 2,
        )(*args, Wnext)
    return pl.pallas_call(
        _node_update_body_last,
        grid=(N // BN,),
        in_specs=ins,
        out_specs=pl.BlockSpec((BN, H), lambda i: (i, 0)),
        out_shape=jax.ShapeDtypeStruct((N, H), jnp.float32),
    )(*args)


def _zero_vec_buf(buf, rows):
    zero = jnp.zeros((16,), jnp.float32)
    cols = buf.shape[1] // 16

    def zrow(r, _):
        for k in range(cols):
            buf[r, pl.ds(k * 16, 16)] = zero
        return 0

    lax.fori_loop(0, rows, zrow, 0)


def _spmem_partition(s):
    base = s * RPS
    nown = jnp.where(s == NS - 1, (N - (NS - 1) * RPS) // C, RPS // C)
    return base, nown


@functools.lru_cache(maxsize=None)
def _sc_layer_kernel():
    mesh = plsc.VectorSubcoreMesh(core_axis_name="c", subcore_axis_name="s")
    NB = 3
    scratch = (
        [pltpu.VMEM((2, C), jnp.int32) for _ in range(NB)]
        + [pltpu.VMEM((C, H), jnp.float32) for _ in range(NB)]
        + [pltpu.VMEM_SHARED((N, H), jnp.float32)]
        + [pltpu.SemaphoreType.DMA] * (2 * NB)
    )

    def body(xs_hbm, eh_hbm, idx_hbm, agg_out, *rest):
        idxs = rest[0:NB]
        ehbs = rest[NB:2 * NB]
        agg_sh = rest[2 * NB]
        sems = rest[2 * NB + 1:]
        slots = [(idxs[b], ehbs[b], sems[2 * b], sems[2 * b + 1])
                 for b in range(NB)]
        c = lax.axis_index("c")
        s = lax.axis_index("s")
        wid = c * NS + s
        base, nown = _spmem_partition(s)

        _zero_vec_buf(ehbs[0], C)

        def zchunk(j, _):
            pltpu.sync_copy(ehbs[0], agg_sh.at[pl.ds(base + j * C, C)])
            return 0

        lax.fori_loop(0, nown, zchunk, 0)
        plsc.subcore_barrier()

        cbase = wid * NCHUNK

        def start(t, sl):
            idxv, ehb, se, sx = sl
            pltpu.sync_copy(idx_hbm.at[cbase + t], idxv)
            pltpu.async_copy(eh_hbm.at[pl.ds((cbase + t) * C, C)], ehb, se)

        def gather(t, sl):
            idxv, ehb, se, sx = sl
            pltpu.make_async_copy(eh_hbm.at[pl.ds(0, C)], ehb, se).wait()
            pltpu.async_copy(xs_hbm.at[idxv.at[0]], ehb, sx, add=True)

        def finish(t, sl, nxt_sl, nxt2_sl):
            idxv, ehb, se, sx = sl
            pltpu.make_async_copy(xs_hbm.at[idxv.at[0]], ehb, sx).wait()

            @pl.when(t + 2 < NCHUNK)
            def _():
                start(t + 2, nxt2_sl)

            @pl.when(t + 1 < NCHUNK)
            def _():
                gather(t + 1, nxt_sl)

            @plsc.parallel_loop(0, C, unroll=4)
            def _row(r):
                for k in range(H // 16):
                    cs = pl.ds(k * 16, 16)
                    ehb[r, cs] = _gelu_sc(ehb[r, cs])

            pltpu.sync_copy(ehb, agg_sh.at[idxv.at[1]], add=True)

        start(0, slots[0])
        start(1, slots[1])
        gather(0, slots[0])

        @pl.loop(0, NCHUNK, step=NB)
        def _main(i3):
            for b in range(NB):
                @pl.when(i3 + b < NCHUNK)
                def _(b=b):
                    finish(i3 + b, slots[b], slots[(b + 1) % NB],
                           slots[(b + 2) % NB])

        plsc.subcore_barrier()

        obase = c * N + base

        def wchunk(j, _):
            pltpu.sync_copy(agg_sh.at[pl.ds(base + j * C, C)], ehbs[0])
            pltpu.sync_copy(ehbs[0], agg_out.at[pl.ds(obase + j * C, C)])
            return 0

        lax.fori_loop(0, nown, wchunk, 0)

    return pl.kernel(
        body, mesh=mesh,
        out_type=[jax.ShapeDtypeStruct((NC * N, H), jnp.float32)],
        scratch_types=scratch)


@functools.lru_cache(maxsize=None)
def _sc_deg_kernel():
    mesh = plsc.VectorSubcoreMesh(core_axis_name="c", subcore_axis_name="s")
    scratch = [
        pltpu.VMEM((C,), jnp.int32),
        pltpu.VMEM((C, H), jnp.float32),
        pltpu.VMEM_SHARED((N, H), jnp.float32),
    ]

    def body(dst_hbm, deg_out, dstv, onesb, deg_sh):
        c = lax.axis_index("c")
        s = lax.axis_index("s")
        wid = c * NS + s
        base, nown = _spmem_partition(s)

        _zero_vec_buf(onesb, C)

        def zchunk(j, _):
            pltpu.sync_copy(onesb, deg_sh.at[pl.ds(base + j * C, C)])
            return 0

        lax.fori_loop(0, nown, zchunk, 0)
        one = jnp.ones((16,), jnp.float32)

        def orow(r, _):
            for k in range(H // 16):
                onesb[r, pl.ds(k * 16, 16)] = one
            return 0

        lax.fori_loop(0, C, orow, 0)
        plsc.subcore_barrier()

        ebase = wid * EPW

        def chunk(i, _):
            off = ebase + i * C
            pltpu.sync_copy(dst_hbm.at[pl.ds(off, C)], dstv)
            pltpu.sync_copy(onesb, deg_sh.at[dstv], add=True)
            return 0

        lax.fori_loop(0, NCHUNK, chunk, 0)
        plsc.subcore_barrier()

        obase = c * N + base

        def wchunk(j, _):
            pltpu.sync_copy(deg_sh.at[pl.ds(base + j * C, C)], onesb)
            pltpu.sync_copy(onesb, deg_out.at[pl.ds(obase + j * C, C)])
            return 0

        lax.fori_loop(0, nown, wchunk, 0)

    return pl.kernel(
        body, mesh=mesh,
        out_type=[jax.ShapeDtypeStruct((NC * N, H), jnp.float32)],
        scratch_types=scratch)


def kernel(node_tokens, edge_tokens, question_tokens, edge_index, edge_batch,
           node_ptr, start_node_locals, start_ptr,
           np_g, np_b, np_W, np_bias, ep_g, ep_b, ep_W, ep_bias,
           l0_Wsrc, l0_Wedge, l0_Wout, l0_g, l0_b,
           l1_Wsrc, l1_Wedge, l1_Wout, l1_g, l1_b):
    src = edge_index[0]
    dst = edge_index[1]
    idxp = jnp.stack([src.reshape(NW, NCHUNK, C), dst.reshape(NW, NCHUNK, C)],
                     axis=2).reshape(NW * NCHUNK, 2, C)

    eh0, eh1 = _edge_prep(edge_tokens, ep_g, ep_b, ep_W, ep_bias,
                          l0_Wedge, l1_Wedge)
    node_h, xs0 = _node_prep(node_tokens, np_g, np_b, np_W, np_bias, l0_Wsrc)

    (degf,) = _sc_deg_kernel()(dst)
    (aggf,) = _sc_layer_kernel()(xs0, eh0, idxp)
    x1, xs1 = _node_update(node_h, aggf[:N], aggf[N:], degf[:N], degf[N:],
                           l0_Wout, l0_g, l0_b, l1_Wsrc)

    (aggf1,) = _sc_layer_kernel()(xs1, eh1, idxp)
    x2 = _node_update(x1, aggf1[:N], aggf1[N:], degf[:N], degf[N:],
                      l1_Wout, l1_g, l1_b, None)
    return x2
